# baseline (speedup 1.0000x reference)
import functools

import jax
import jax.numpy as jnp
from jax import lax
from jax.experimental import pallas as pl
from jax.experimental.pallas import tpu as pltpu
from jax.experimental.pallas import tpu_sc as plsc

B = 16384
V = 1000000
D = 32
NEG = 10

NC = 2
NS = 16
NW = NC * NS

CHUNK = 128
U_PER_W = B // NW
UC = U_PER_W // CHUNK
NC_PER_W = (B * NEG) // NW
NCC = NC_PER_W // CHUNK

TB = 512
NT = B // TB
ROWS_PER_TILE = TB * (1 + NEG)


def _sc_gather(uidx2, pidx2, nidx2, user_table, item_table, neg_table):
    mesh = plsc.VectorSubcoreMesh(core_axis_name="c", subcore_axis_name="s")

    @functools.partial(
        pl.kernel,
        mesh=mesh,
        compiler_params=pltpu.CompilerParams(use_tc_tiling_on_sc=False),
        out_type=[
            jax.ShapeDtypeStruct((B, D), jnp.float32),
            jax.ShapeDtypeStruct((B, D), jnp.float32),
            jax.ShapeDtypeStruct((B * NEG, D), jnp.float32),
        ],
        scratch_types=[
            pltpu.VMEM((UC, CHUNK), jnp.int32),
            pltpu.VMEM((UC, CHUNK), jnp.int32),
            pltpu.VMEM((NCC, CHUNK), jnp.int32),
            pltpu.VMEM((CHUNK, D), jnp.float32),
            pltpu.SemaphoreType.DMA,
        ],
    )
    def k(uidx_h, pidx_h, nidx_h, ut_h, it_h, nt_h, uout, pout, nout,
          uidx_v, pidx_v, nidx_v, rows_v, sem):
        wid = lax.axis_index("s") * NC + lax.axis_index("c")
        pltpu.sync_copy(uidx_h.at[pl.ds(wid * UC, UC)], uidx_v)
        pltpu.sync_copy(pidx_h.at[pl.ds(wid * UC, UC)], pidx_v)
        pltpu.sync_copy(nidx_h.at[pl.ds(wid * NCC, NCC)], nidx_v)

        ubase = wid * U_PER_W
        for j in range(UC):
            pltpu.async_copy(ut_h.at[uidx_v.at[j]], rows_v, sem).wait()
            pltpu.sync_copy(rows_v, uout.at[pl.ds(ubase + j * CHUNK, CHUNK)])
        for j in range(UC):
            pltpu.async_copy(it_h.at[pidx_v.at[j]], rows_v, sem).wait()
            pltpu.sync_copy(rows_v, pout.at[pl.ds(ubase + j * CHUNK, CHUNK)])

        nbase = wid * NC_PER_W

        def body(j, carry):
            pltpu.async_copy(nt_h.at[nidx_v.at[j]], rows_v, sem).wait()
            pltpu.sync_copy(rows_v, nout.at[pl.ds(nbase + j * CHUNK, CHUNK)])
            return carry

        lax.fori_loop(0, NCC, body, 0)

    return k(uidx2, pidx2, nidx2, user_table, item_table, neg_table)


def _tc_body(u_ref, p_ref, n_ref, w1_ref, b1_ref, w2_ref, b2_ref,
             w3_ref, b3_ref, wdg_ref, wdm_ref, bd_ref, out_ref):
    u = u_ref[...]
    p = p_ref[...]
    n = n_ref[...]
    ut = jnp.broadcast_to(u[:, None, :], (TB, NEG, D)).reshape(TB * NEG, D)

    users = jnp.concatenate([u, ut], axis=0)
    items = jnp.concatenate([p, n], axis=0)

    x = jnp.concatenate([users, items], axis=1)
    h = jnp.maximum(jnp.dot(x, w1_ref[...], preferred_element_type=jnp.float32)
                    + b1_ref[...], 0.0)
    h = jnp.maximum(jnp.dot(h, w2_ref[...], preferred_element_type=jnp.float32)
                    + b2_ref[...], 0.0)
    h = jnp.maximum(jnp.dot(h, w3_ref[...], preferred_element_type=jnp.float32)
                    + b3_ref[...], 0.0)

    g = jax.nn.sigmoid(users * items)

    logit = (jnp.sum(g * wdg_ref[...], axis=1, keepdims=True)
             + jnp.sum(h * wdm_ref[...], axis=1, keepdims=True)
             + bd_ref[0, 0])
    out_ref[...] = logit


def _tc_mlp(user_rows, pos_rows, neg_rows2, W1, b1, W2, b2, W3, b3, wdg, wdm, bd):
    full = lambda shape: pl.BlockSpec(shape, lambda i: (0, 0))
    return pl.pallas_call(
        _tc_body,
        grid=(NT,),
        in_specs=[
            pl.BlockSpec((TB, D), lambda i: (i, 0)),
            pl.BlockSpec((TB, D), lambda i: (i, 0)),
            pl.BlockSpec((TB * NEG, D), lambda i: (i, 0)),
            full((2 * D, 64)), full((1, 64)),
            full((64, 16)), full((1, 16)),
            full((16, 8)), full((1, 8)),
            full((1, D)), full((1, 8)), full((1, 1)),
        ],
        out_specs=pl.BlockSpec((ROWS_PER_TILE, 1), lambda i: (i, 0)),
        out_shape=jax.ShapeDtypeStruct((NT * ROWS_PER_TILE, 1), jnp.float32),
    )(user_rows, pos_rows, neg_rows2, W1, b1, W2, b2, W3, b3, wdg, wdm, bd)


def kernel(user_inputs, pos_inputs, neg_inputs, user_table, item_table,
           neg_item_table, W1, b1, W2, b2, W3, b3, Wd, bd):
    uidx2 = user_inputs.reshape(B // CHUNK, CHUNK).astype(jnp.int32)
    pidx2 = pos_inputs.reshape(B // CHUNK, CHUNK).astype(jnp.int32)
    nidx2 = neg_inputs.reshape(B * NEG // CHUNK, CHUNK).astype(jnp.int32)

    user_rows, pos_rows, neg_rows = _sc_gather(
        uidx2, pidx2, nidx2, user_table, item_table, neg_item_table)

    wdg = Wd[:D].reshape(1, D)
    wdm = Wd[D:].reshape(1, 8)
    out = _tc_mlp(user_rows, pos_rows, neg_rows,
                  W1, b1.reshape(1, 64), W2, b2.reshape(1, 16),
                  W3, b3.reshape(1, 8), wdg, wdm, bd.reshape(1, 1))

    o = out.reshape(NT, ROWS_PER_TILE)
    pos_log = o[:, :TB].reshape(B, 1)
    neg_log = o[:, TB:].reshape(B, NEG)
    return jnp.concatenate([pos_log, neg_log], axis=1)
